# Initial kernel scaffold; baseline (speedup 1.0000x reference)
import jax
import jax.numpy as jnp
from jax.experimental import pallas as pl


def kernel(A_input, A_week_input, A_weekend_input, Location_location_input, Temporal_input, Temporal_week_input, Temporal_weekend_input, Distance_input, Duration_input, Location_time_input, W1i_t, W1r_t, b1_t, W2i_t, W2r_t, b2_t, W1i_w, W1r_w, b1_w, W2i_w, W2r_w, b2_w, W1i_e, W1r_e, b1_e, W2i_e, W2r_e, b2_e, W1i_d, W1r_d, b1_d, W2i_d, W2r_d, b2_d, W1i_u, W1r_u, b1_u, W2i_u, W2r_u, b2_u, W1i_l, W1r_l, b1_l, W2i_l, W2r_l, b2_l, W_lt, b_lt, W_l2, b_l2, W_dense, b_dense, W_gnn, b_gnn):
    raise NotImplementedError("write your pallas kernel here")


# speedup vs baseline: 160.1541x; 160.1541x over previous
import jax
import jax.numpy as jnp
from jax import lax
from jax.experimental import pallas as pl
from jax.experimental.pallas import tpu as pltpu
from jax.experimental.pallas import tpu_sc as plsc

N = 50000
E = 800000
B = 2

NB = 2048
GRID = -(-N // NB)
NB1 = 1024
GRID1 = -(-N // NB1)
NBS = 1000
GRIDS = N // NBS

_NC, _NS = 2, 16
_NT = _NC * _NS
_CH = 128
_EPT = (E // _NT) // _CH * _CH
_NFULL = _EPT // _CH
_TAIL_BASE = _NT * _EPT
_NEXTRA = (E - _TAIL_BASE) // _CH
_RPT = 3128
_RPT_LAST = N - (_NS - 1) * _RPT


def _mesh():
    return plsc.VectorSubcoreMesh(core_axis_name="c", subcore_axis_name="s")


def _sc_message_pass(src, dst, table, width):
    def body(src_h, dst_h, tab_h, zeros_h, out_h, idx_s, idx_d, rows, acc, sem):
        c = lax.axis_index("c")
        s = lax.axis_index("s")
        wid = c * _NS + s
        pltpu.sync_copy(zeros_h, acc.at[pl.ds(s * _RPT, _RPT)])
        plsc.subcore_barrier()
        nch = jnp.where(wid < _NEXTRA, _NFULL + 1, _NFULL)

        def step(i, carry):
            base = jnp.where(i < _NFULL,
                             wid * _EPT + i * _CH,
                             _TAIL_BASE + wid * _CH)
            pltpu.sync_copy(src_h.at[pl.ds(base, _CH)], idx_s)
            pltpu.sync_copy(dst_h.at[pl.ds(base, _CH)], idx_d)
            pltpu.async_copy(tab_h.at[idx_s], rows, sem).wait()
            pltpu.sync_copy(rows, acc.at[idx_d], add=True)
            return carry

        lax.fori_loop(0, nch, step, 0)
        plsc.subcore_barrier()
        pltpu.sync_copy(acc.at[pl.ds(s * _RPT, _RPT)],
                        out_h.at[c, pl.ds(s * _RPT, _RPT)])

    k = pl.kernel(
        body,
        out_type=jax.ShapeDtypeStruct((_NC, N, width), jnp.float32),
        mesh=_mesh(),
        compiler_params=pltpu.CompilerParams(use_tc_tiling_on_sc=False),
        scratch_types=[
            pltpu.VMEM((_CH,), jnp.int32),
            pltpu.VMEM((_CH,), jnp.int32),
            pltpu.VMEM((_CH, width), jnp.float32),
            pltpu.VMEM_SHARED((N, width), jnp.float32),
            pltpu.SemaphoreType.DMA,
        ],
    )
    zeros = jnp.zeros((_RPT, width), jnp.float32)
    return k(src, dst, table, zeros)


def _sc_degrees(dsts):
    def body(d0, d1, d2, d3, ones_h, zeros_h, out_h,
             ones_v, idx_d, a0, a1, a2, a3):
        c = lax.axis_index("c")
        s = lax.axis_index("s")
        wid = c * _NS + s
        pltpu.sync_copy(ones_h, ones_v)
        for acc in (a0, a1, a2, a3):
            pltpu.sync_copy(zeros_h, acc.at[pl.ds(s * _RPT, _RPT)])
        plsc.subcore_barrier()
        nch = jnp.where(wid < _NEXTRA, _NFULL + 1, _NFULL)

        for dh, acc in zip((d0, d1, d2, d3), (a0, a1, a2, a3)):
            def step(i, carry, dh=dh, acc=acc):
                base = jnp.where(i < _NFULL,
                                 wid * _EPT + i * _CH,
                                 _TAIL_BASE + wid * _CH)
                pltpu.sync_copy(dh.at[pl.ds(base, _CH)], idx_d)
                pltpu.sync_copy(ones_v, acc.at[idx_d], add=True)
                return carry

            lax.fori_loop(0, nch, step, 0)
        plsc.subcore_barrier()
        for l, acc in enumerate((a0, a1, a2, a3)):
            pltpu.sync_copy(acc.at[pl.ds(s * _RPT, _RPT)],
                            out_h.at[c, l, pl.ds(s * _RPT, _RPT)])

    k = pl.kernel(
        body,
        out_type=jax.ShapeDtypeStruct((_NC, 4, N), jnp.float32),
        mesh=_mesh(),
        compiler_params=pltpu.CompilerParams(use_tc_tiling_on_sc=False),
        scratch_types=[
            pltpu.VMEM((_CH,), jnp.float32),
            pltpu.VMEM((_CH,), jnp.int32),
            pltpu.VMEM_SHARED((N,), jnp.float32),
            pltpu.VMEM_SHARED((N,), jnp.float32),
            pltpu.VMEM_SHARED((N,), jnp.float32),
            pltpu.VMEM_SHARED((N,), jnp.float32),
        ],
    )
    ones = jnp.ones((_CH,), jnp.float32)
    zeros = jnp.zeros((_RPT,), jnp.float32)
    return k(dsts[0], dsts[1], dsts[2], dsts[3], ones, zeros)


def _dot(a, b):
    return jnp.dot(a, b, preferred_element_type=jnp.float32)


def _gelu_exact(x):
    return 0.5 * x * (1.0 + lax.erf(x * (2.0 ** -0.5)))


def _elu(x):
    return jnp.where(x > 0, x, jnp.exp(jnp.minimum(x, 0.0)) - 1.0)


def _softmax0(x):
    m = jnp.max(x, axis=0, keepdims=True)
    e = jnp.exp(x - m)
    return e / jnp.sum(e, axis=0, keepdims=True)


def _tc_prep(xs, degs, w1i, w1r, b1, w_lt, b_lt, w_l2, b_l2):
    n_br = 6

    def body(*refs):
        (xt, xw, xe, xd, xu, xl, dg,
         wit, wiw, wie, wid_, wiu, wil,
         wrt, wrw, wre, wrd, wru, wrl,
         bt, bw, be, bd, bu, bl_,
         wlt, blt, wl2, bl2,
         dinv_o, t1t, t1w, t1e, t1d, t1u, t1l,
         r1t, r1w, r1e, r1d, r1u, r1l, olt_o) = refs
        deg = dg[0] + dg[1]
        dinv = jnp.where(deg > 0, lax.rsqrt(deg), 0.0)
        dinv_o[...] = dinv.T
        dv_by_list = (dinv[0][:, None], dinv[1][:, None],
                      dinv[2][:, None], dinv[3][:, None])
        lists = (0, 1, 2, 0, 0, 3)
        xs_ = (xt, xw, xe, xd, xu, xl)
        wis = (wit, wiw, wie, wid_, wiu, wil)
        wrs = (wrt, wrw, wre, wrd, wru, wrl)
        bs = (bt, bw, be, bd, bu, bl_)
        t1s = (t1t, t1w, t1e, t1d, t1u, t1l)
        r1s = (r1t, r1w, r1e, r1d, r1u, r1l)
        for i in range(n_br):
            x0 = xs_[i][0]
            x1 = xs_[i][1]
            dv = dv_by_list[lists[i]]
            h = jnp.concatenate([_dot(x0, wis[i][...]),
                                 _dot(x1, wis[i][...])], axis=1)
            t1s[i][...] = h * dv
            r1s[i][...] = jnp.concatenate([_dot(x0, wrs[i][...]),
                                           _dot(x1, wrs[i][...])],
                                          axis=1) + bs[i][...]
        l0 = xl[0]
        l1 = xl[1]
        o0 = _dot(_dot(l0, wlt[...]) + blt[...], wl2[...])
        o1 = _dot(_dot(l1, wlt[...]) + blt[...], wl2[...])
        olt_o[...] = jnp.concatenate([o0, o1], axis=1) + bl2[...]

    x_specs = [pl.BlockSpec((B, NB1, x.shape[2]), lambda i: (0, i, 0))
               for x in xs]
    deg_spec = pl.BlockSpec((2, 4, NB1), lambda i: (0, 0, i))
    w_specs = ([pl.BlockSpec(w.shape, lambda i: (0, 0)) for w in w1i]
               + [pl.BlockSpec(w.shape, lambda i: (0, 0)) for w in w1r]
               + [pl.BlockSpec(b.shape, lambda i: (0, 0)) for b in b1]
               + [pl.BlockSpec(w_lt.shape, lambda i: (0, 0)),
                  pl.BlockSpec(b_lt.shape, lambda i: (0, 0)),
                  pl.BlockSpec(w_l2.shape, lambda i: (0, 0)),
                  pl.BlockSpec(b_l2.shape, lambda i: (0, 0))])
    out_shape = ([jax.ShapeDtypeStruct((N, 4), jnp.float32)]
                 + [jax.ShapeDtypeStruct((N, 40), jnp.float32)] * 12
                 + [jax.ShapeDtypeStruct((N, 24), jnp.float32)])
    out_specs = ([pl.BlockSpec((NB1, 4), lambda i: (i, 0))]
                 + [pl.BlockSpec((NB1, 40), lambda i: (i, 0))] * 12
                 + [pl.BlockSpec((NB1, 24), lambda i: (i, 0))])
    return pl.pallas_call(
        body,
        grid=(GRID1,),
        in_specs=x_specs + [deg_spec] + w_specs,
        out_specs=out_specs,
        out_shape=out_shape,
    )(*xs, degs, *w1i, *w1r, *b1, w_lt, b_lt, w_l2, b_l2)


def _tc_mid(parts, xr1s, dinvT, bd2i, bd2r, b2):
    lists = (0, 1, 2, 0, 0, 3)

    def body(*refs):
        ps = refs[0:6]
        rs = refs[6:12]
        dv_all = refs[12]
        wis = refs[13:19]
        wrs = refs[19:25]
        bs = refs[25:31]
        t2s = refs[31:37]
        r2s = refs[37:43]
        for i in range(6):
            dv = dv_all[:, lists[i]][:, None]
            p = ps[i]
            agg = dv * (p[0] + p[1]) + rs[i][...]
            out1 = _elu(_gelu_exact(agg))
            t2s[i][...] = dv * _dot(out1, wis[i][...])
            r2s[i][...] = _dot(out1, wrs[i][...]) + bs[i][...]

    p_spec = pl.BlockSpec((2, NBS, 40), lambda i: (0, i, 0))
    r_spec = pl.BlockSpec((NBS, 40), lambda i: (i, 0))
    dv_spec = pl.BlockSpec((NBS, 4), lambda i: (i, 0))
    w_spec = pl.BlockSpec((40, 24), lambda i: (0, 0))
    b_spec = pl.BlockSpec((1, 24), lambda i: (0, 0))
    out_shape = [jax.ShapeDtypeStruct((N, 24), jnp.float32)] * 12
    out_specs = [pl.BlockSpec((NBS, 24), lambda i: (i, 0))] * 12
    return pl.pallas_call(
        body,
        grid=(GRIDS,),
        in_specs=([p_spec] * 6 + [r_spec] * 6 + [dv_spec]
                  + [w_spec] * 12 + [b_spec] * 6),
        out_specs=out_specs,
        out_shape=out_shape,
    )(*parts, *xr1s, dinvT, *bd2i, *bd2r, *b2)


def _acc_update(ref, val, op):
    i = pl.program_id(0)

    @pl.when(i == 0)
    def _():
        ref[...] = val

    @pl.when(i > 0)
    def _():
        ref[...] = op(ref[...], val)


_S_SPEC = lambda: pl.BlockSpec((NBS, 24), lambda i: (i, 0))
_M_SPEC = lambda: pl.BlockSpec((8, 24), lambda i: (0, 0))
_S_OUT = lambda: jax.ShapeDtypeStruct((N, 24), jnp.float32)
_M_OUT = lambda: jax.ShapeDtypeStruct((8, 24), jnp.float32)
_LISTS = (0, 1, 2, 0, 0, 3)


def _tc_pre(parts2, xr2s, dinvT, olt_pre):
    def body(*refs):
        ps = refs[0:6]
        rs = refs[6:12]
        dvr = refs[12]
        op = refs[13]
        outs = refs[14:20]
        mxr = refs[20]
        ms = []
        for b in range(6):
            dv = dvr[:, _LISTS[b]][:, None]
            h = jnp.maximum(dv * (ps[b][0] + ps[b][1]) + rs[b][...], 0.0)
            outs[b][...] = h
            ms.append(jnp.max(h, axis=0)[None])
        ms.append(jnp.max(op[...], axis=0)[None])
        ms.append(jnp.full((1, 24), -jnp.inf, jnp.float32))
        _acc_update(mxr, jnp.concatenate(ms, axis=0), jnp.maximum)

    p_spec = pl.BlockSpec((2, NBS, 24), lambda i: (0, i, 0))
    dv_spec = pl.BlockSpec((NBS, 4), lambda i: (i, 0))
    return pl.pallas_call(
        body,
        grid=(GRIDS,),
        in_specs=[p_spec] * 6 + [_S_SPEC()] * 6 + [dv_spec, _S_SPEC()],
        out_specs=[_S_SPEC()] * 6 + [_M_SPEC()],
        out_shape=[_S_OUT()] * 6 + [_M_OUT()],
    )(*parts2, *xr2s, dinvT, olt_pre)


def _tc_exp(pres, olt_pre, mx):
    def body(*refs):
        ins = refs[0:7]
        mxr = refs[7]
        outs = refs[8:15]
        smr = refs[15]
        m = mxr[...]
        ss = []
        for b in range(7):
            e = jnp.exp(ins[b][...] - m[b][None])
            outs[b][...] = e
            ss.append(jnp.sum(e, axis=0)[None])
        ss.append(jnp.ones((1, 24), jnp.float32))
        _acc_update(smr, jnp.concatenate(ss, axis=0), lax.add)

    return pl.pallas_call(
        body,
        grid=(GRIDS,),
        in_specs=[_S_SPEC()] * 7 + [_M_SPEC()],
        out_specs=[_S_SPEC()] * 7 + [_M_SPEC()],
        out_shape=[_S_OUT()] * 7 + [_M_OUT()],
    )(*pres, olt_pre, mx)


def _tc_heads(es, sm, bd_gnn, b_gnn, bd_dense, b_dense):
    def body(et, ew, ee, ed_, eu, el, eo, smr, wg, bg, wd, bd, gp_o, dp_o, mxr):
        inv = 1.0 / smr[...]
        g = (et[...] * inv[0][None] + ew[...] * inv[1][None]
             + ee[...] * inv[2][None] + ed_[...] * inv[3][None]
             + eu[...] * inv[4][None])
        gp = _dot(g, wg[...]) + bg[...]
        dp = _dot(2.0 * el[...] * inv[5][None] + 2.0 * eo[...] * inv[6][None],
                  wd[...]) + bd[...]
        gp_o[...] = gp
        dp_o[...] = dp
        pad = jnp.full((6, 24), -jnp.inf, jnp.float32)
        val = jnp.concatenate([jnp.max(gp, axis=0)[None],
                               jnp.max(dp, axis=0)[None], pad], axis=0)
        _acc_update(mxr, val, jnp.maximum)

    w_spec = pl.BlockSpec((24, 24), lambda i: (0, 0))
    b_spec = pl.BlockSpec((1, 24), lambda i: (0, 0))
    return pl.pallas_call(
        body,
        grid=(GRIDS,),
        in_specs=[_S_SPEC()] * 7 + [_M_SPEC(), w_spec, b_spec, w_spec, b_spec],
        out_specs=[_S_SPEC(), _S_SPEC(), _M_SPEC()],
        out_shape=[_S_OUT(), _S_OUT(), _M_OUT()],
    )(*es, sm, bd_gnn, b_gnn, bd_dense, b_dense)


def _tc_heads_exp(gp, dp, mx2):
    def body(gpr, dpr, mxr, eg_o, ed_o, smr):
        m = mxr[...]
        eg = jnp.exp(gpr[...] - m[0][None])
        ed = jnp.exp(dpr[...] - m[1][None])
        eg_o[...] = eg
        ed_o[...] = ed
        val = jnp.concatenate([jnp.sum(eg, axis=0)[None],
                               jnp.sum(ed, axis=0)[None],
                               jnp.ones((6, 24), jnp.float32)], axis=0)
        _acc_update(smr, val, lax.add)

    return pl.pallas_call(
        body,
        grid=(GRIDS,),
        in_specs=[_S_SPEC(), _S_SPEC(), _M_SPEC()],
        out_specs=[_S_SPEC(), _S_SPEC(), _M_SPEC()],
        out_shape=[_S_OUT(), _S_OUT(), _M_OUT()],
    )(gp, dp, mx2)


def _tc_final(eg, ed, sm2):
    def body(egr, edr, smr, out):
        inv = 1.0 / smr[...]
        out[...] = egr[...] * inv[0][None] + edr[...] * inv[1][None]

    return pl.pallas_call(
        body,
        grid=(GRIDS,),
        in_specs=[_S_SPEC(), _S_SPEC(), _M_SPEC()],
        out_specs=_S_SPEC(),
        out_shape=_S_OUT(),
    )(eg, ed, sm2)


def _bd(w):
    z = jnp.zeros_like(w)
    return jnp.block([[w, z], [z, w]])


def kernel(A_input, A_week_input, A_weekend_input, Location_location_input,
           Temporal_input, Temporal_week_input, Temporal_weekend_input,
           Distance_input, Duration_input, Location_time_input,
           W1i_t, W1r_t, b1_t, W2i_t, W2r_t, b2_t,
           W1i_w, W1r_w, b1_w, W2i_w, W2r_w, b2_w,
           W1i_e, W1r_e, b1_e, W2i_e, W2r_e, b2_e,
           W1i_d, W1r_d, b1_d, W2i_d, W2r_d, b2_d,
           W1i_u, W1r_u, b1_u, W2i_u, W2r_u, b2_u,
           W1i_l, W1r_l, b1_l, W2i_l, W2r_l, b2_l,
           W_lt, b_lt, W_l2, b_l2, W_dense, b_dense, W_gnn, b_gnn):
    ea = A_input.reshape(2, -1)
    ew = A_week_input.reshape(2, -1)
    ee = A_weekend_input.reshape(2, -1)
    el = Location_location_input.reshape(2, -1)
    srcs = (ea[0], ew[0], ee[0], el[0])
    dsts = (ea[1], ew[1], ee[1], el[1])

    degs = _sc_degrees(list(dsts))

    xs = (Temporal_input, Temporal_week_input, Temporal_weekend_input,
          Distance_input, Duration_input, Location_time_input)
    w1i = (W1i_t, W1i_w, W1i_e, W1i_d, W1i_u, W1i_l)
    w1r = (W1r_t, W1r_w, W1r_e, W1r_d, W1r_u, W1r_l)
    b1 = tuple(jnp.tile(b, 2)[None] for b in
               (b1_t, b1_w, b1_e, b1_d, b1_u, b1_l))
    prep = _tc_prep(xs, degs, w1i, w1r, b1,
                    W_lt, b_lt[None],
                    W_l2, jnp.tile(b_l2, 2)[None])
    dinvT = prep[0]
    t1s = prep[1:7]
    xr1s = prep[7:13]
    olt_pre = prep[13]

    br_list = (0, 1, 2, 0, 0, 3)
    parts1 = [_sc_message_pass(srcs[br_list[i]], dsts[br_list[i]],
                               t1s[i], 40) for i in range(6)]

    bd2i = tuple(_bd(w) for w in (W2i_t, W2i_w, W2i_e, W2i_d, W2i_u, W2i_l))
    bd2r = tuple(_bd(w) for w in (W2r_t, W2r_w, W2r_e, W2r_d, W2r_u, W2r_l))
    b2 = tuple(jnp.tile(b, 2)[None] for b in
               (b2_t, b2_w, b2_e, b2_d, b2_u, b2_l))
    mid = _tc_mid(parts1, xr1s, dinvT, bd2i, bd2r, b2)
    t2s = mid[0:6]
    xr2s = mid[6:12]

    parts2 = [_sc_message_pass(srcs[br_list[i]], dsts[br_list[i]],
                               t2s[i], 24) for i in range(6)]

    pre = _tc_pre(parts2, xr2s, dinvT, olt_pre)
    ex = _tc_exp(pre[0:6], olt_pre, pre[6])
    gp, dp, mx2 = _tc_heads(ex[0:7], ex[7],
                            _bd(W_gnn), jnp.tile(b_gnn, 2)[None],
                            _bd(W_dense), jnp.tile(b_dense, 2)[None])
    eg, ed, sm2 = _tc_heads_exp(gp, dp, mx2)
    out2d = _tc_final(eg, ed, sm2)
    out = out2d.reshape(N, B, 12)
    return jnp.transpose(out, (1, 2, 0))
